# baseline (speedup 1.0000x reference)
import functools

import jax
import jax.numpy as jnp
from jax.experimental import pallas as pl
from jax.experimental.pallas import tpu as pltpu

BLK = 2048
NEG = -1e30


def _fused_kernel(seg_ref, flat_ref, extra_ref, W1a_ref, W1b_ref, b1_ref,
                  W2_ref, b2_ref, W3_ref, b3_ref, Wf1_ref, bf1_ref,
                  Wf2_ref, bf2_ref, out_ref,
                  sb_ref, m_ref, s_ref, acc_ref, *, nsteps, nseg):
    i = pl.program_id(0)

    @pl.when(i == 0)
    def _init():
        sb_ref[...] = (
            jnp.dot(extra_ref[...], W1b_ref[...],
                    preferred_element_type=jnp.float32) + b1_ref[...])
        m_ref[...] = jnp.full_like(m_ref, NEG)
        s_ref[...] = jnp.zeros_like(s_ref)
        acc_ref[...] = jnp.zeros_like(acc_ref)

    x = flat_ref[...]
    sid = seg_ref[0]
    oh_bool = (jax.lax.broadcasted_iota(jnp.int32, (BLK, nseg), 1) == sid)
    oh = oh_bool.astype(jnp.float32)

    h1 = jnp.dot(x.astype(jnp.bfloat16), W1a_ref[...].astype(jnp.bfloat16),
                 preferred_element_type=jnp.float32)
    h1 = h1 + jnp.dot(oh, sb_ref[...], preferred_element_type=jnp.float32)
    h1 = jnp.maximum(h1, 0.0)
    h2 = jnp.maximum(
        jnp.dot(h1.astype(jnp.bfloat16), W2_ref[...].astype(jnp.bfloat16),
                preferred_element_type=jnp.float32)
        + b2_ref[...], 0.0)
    l_col = (jnp.dot(h2, W3_ref[...], preferred_element_type=jnp.float32)
             + b3_ref[...])

    masked = jnp.where(oh_bool, l_col, NEG)
    bm_row = jnp.max(masked, axis=0, keepdims=True)
    m_old = m_ref[...]
    new_m = jnp.maximum(m_old, bm_row)
    scale_row = jnp.exp(m_old - new_m)
    m_ref[...] = new_m

    nm_tok = jnp.sum(oh * new_m, axis=1, keepdims=True)
    e_col = jnp.exp(l_col - nm_tok)
    s_ref[...] = (s_ref[...] * scale_row
                  + jnp.sum(oh * e_col, axis=0, keepdims=True))

    eye = (jax.lax.broadcasted_iota(jnp.int32, (nseg, nseg), 0)
           == jax.lax.broadcasted_iota(jnp.int32, (nseg, nseg), 1))
    scale_col = jnp.sum(jnp.where(eye, scale_row, 0.0), axis=1,
                        keepdims=True)
    ex_oh = oh * e_col
    blk_acc = jax.lax.dot_general(
        ex_oh, x, (((0,), (0,)), ((), ())),
        preferred_element_type=jnp.float32)
    acc_ref[...] = acc_ref[...] * scale_col + blk_acc

    @pl.when(i == nsteps - 1)
    def _final():
        s_col = jnp.sum(jnp.where(eye, s_ref[...], 0.0), axis=1,
                        keepdims=True)
        pooled = jnp.where(s_col > 0.0, acc_ref[...] / s_col, 0.0)
        g = jnp.maximum(
            jnp.dot(pooled, Wf1_ref[...], preferred_element_type=jnp.float32)
            + bf1_ref[...], 0.0)
        out_ref[...] = (jnp.dot(g, Wf2_ref[...],
                                preferred_element_type=jnp.float32)
                        + bf2_ref[...])


@jax.jit
def kernel(io_embed, flat, segment_ids, W1, b1, W2, b2, W3, b3,
           Wf1, bf1, Wf2, bf2):
    B, NIO, D = io_embed.shape
    T = flat.shape[0]
    nsteps = T // BLK
    extra = io_embed.reshape(B, NIO * D)
    W1a = W1[:D]
    W1b = W1[D:]
    seg3 = segment_ids.reshape(nsteps, BLK, 1)

    grid = (nsteps,)

    out = pl.pallas_call(
        functools.partial(_fused_kernel, nsteps=nsteps, nseg=B),
        grid=grid,
        in_specs=[
            pl.BlockSpec((1, BLK, 1), lambda i: (i, 0, 0)),
            pl.BlockSpec((BLK, D), lambda i: (i, 0)),
            pl.BlockSpec((B, NIO * D), lambda i: (0, 0)),
            pl.BlockSpec((D, D), lambda i: (0, 0)),
            pl.BlockSpec((NIO * D, D), lambda i: (0, 0)),
            pl.BlockSpec((1, D), lambda i: (0, 0)),
            pl.BlockSpec((D, D), lambda i: (0, 0)),
            pl.BlockSpec((1, D), lambda i: (0, 0)),
            pl.BlockSpec((D, 1), lambda i: (0, 0)),
            pl.BlockSpec((1, 1), lambda i: (0, 0)),
            pl.BlockSpec((D, D), lambda i: (0, 0)),
            pl.BlockSpec((1, D), lambda i: (0, 0)),
            pl.BlockSpec((D, 2), lambda i: (0, 0)),
            pl.BlockSpec((1, 2), lambda i: (0, 0)),
        ],
        out_specs=pl.BlockSpec((B, 2), lambda i: (0, 0)),
        out_shape=jax.ShapeDtypeStruct((B, 2), jnp.float32),
        scratch_shapes=[
            pltpu.VMEM((B, D), jnp.float32),
            pltpu.VMEM((1, B), jnp.float32),
            pltpu.VMEM((1, B), jnp.float32),
            pltpu.VMEM((B, D), jnp.float32),
        ],
    )(seg3, flat, extra, W1a, W1b, b1.reshape(1, D), W2, b2.reshape(1, D),
      W3, b3.reshape(1, 1), Wf1, bf1.reshape(1, D), Wf2, bf2.reshape(1, 2))
    return out


# speedup vs baseline: 1.0643x; 1.0643x over previous
import functools

import jax
import jax.numpy as jnp
from jax.experimental import pallas as pl
from jax.experimental.pallas import tpu as pltpu

BLK = 2048
NEG = -1e30


def _fused_kernel(seg_ref, flat_ref, extra_ref, W1a_ref, W1b_ref, b1_ref,
                  W2_ref, b2_ref, W3_ref, b3_ref, Wf1_ref, bf1_ref,
                  Wf2_ref, bf2_ref, out_ref,
                  sb_ref, m_ref, s_ref, acc_ref, *, nsteps, nseg):
    i = pl.program_id(0)

    @pl.when(i == 0)
    def _init():
        sb_ref[...] = (
            jnp.dot(extra_ref[...], W1b_ref[...],
                    preferred_element_type=jnp.float32) + b1_ref[...])
        m_ref[...] = jnp.full_like(m_ref, NEG)
        s_ref[...] = jnp.zeros_like(s_ref)
        acc_ref[...] = jnp.zeros_like(acc_ref)

    x = flat_ref[...]
    sid = seg_ref[0]
    oh_bool = (jax.lax.broadcasted_iota(jnp.int32, (BLK, nseg), 1) == sid)
    oh = oh_bool.astype(jnp.float32)

    h1 = jnp.dot(x, W1a_ref[...], preferred_element_type=jnp.float32)
    h1 = h1 + jnp.dot(oh, sb_ref[...], preferred_element_type=jnp.float32)
    h1 = jnp.maximum(h1, 0.0)
    h2 = jnp.maximum(
        jnp.dot(h1, W2_ref[...], preferred_element_type=jnp.float32)
        + b2_ref[...], 0.0)
    L = (jnp.dot(h2, W3_ref[...], preferred_element_type=jnp.float32)
         + b3_ref[...])

    masked = jnp.where(oh_bool, L, NEG)
    bm_row = jnp.max(masked, axis=0, keepdims=True)
    m_old = m_ref[...]
    new_m = jnp.maximum(m_old, bm_row)
    scale_row = jnp.exp(m_old - new_m)
    m_ref[...] = new_m

    E = jnp.where(oh_bool, jnp.exp(L - new_m), 0.0)
    s_ref[...] = (s_ref[...] * scale_row
                  + jnp.sum(E, axis=0, keepdims=True))

    eye = (jax.lax.broadcasted_iota(jnp.int32, (nseg, nseg), 0)
           == jax.lax.broadcasted_iota(jnp.int32, (nseg, nseg), 1))
    scale_col = jnp.sum(jnp.where(eye, scale_row, 0.0), axis=1,
                        keepdims=True)
    blk_acc = jax.lax.dot_general(
        E, x, (((0,), (0,)), ((), ())),
        preferred_element_type=jnp.float32)
    acc_ref[...] = acc_ref[...] * scale_col + blk_acc

    @pl.when(i == nsteps - 1)
    def _final():
        s_col = jnp.sum(jnp.where(eye, s_ref[...], 0.0), axis=1,
                        keepdims=True)
        pooled = jnp.where(s_col > 0.0, acc_ref[...] / s_col, 0.0)
        g = jnp.maximum(
            jnp.dot(pooled, Wf1_ref[...], preferred_element_type=jnp.float32)
            + bf1_ref[...], 0.0)
        out_ref[...] = (jnp.dot(g, Wf2_ref[...],
                                preferred_element_type=jnp.float32)
                        + bf2_ref[...])


@jax.jit
def kernel(io_embed, flat, segment_ids, W1, b1, W2, b2, W3, b3,
           Wf1, bf1, Wf2, bf2):
    B, NIO, D = io_embed.shape
    T = flat.shape[0]
    nsteps = T // BLK
    extra = io_embed.reshape(B, NIO * D)
    W1a = W1[:D]
    W1b = W1[D:]
    seg3 = segment_ids.reshape(nsteps, BLK, 1)

    grid = (nsteps,)

    out = pl.pallas_call(
        functools.partial(_fused_kernel, nsteps=nsteps, nseg=B),
        grid=grid,
        in_specs=[
            pl.BlockSpec((1, BLK, 1), lambda i: (i, 0, 0)),
            pl.BlockSpec((BLK, D), lambda i: (i, 0)),
            pl.BlockSpec((B, NIO * D), lambda i: (0, 0)),
            pl.BlockSpec((D, D), lambda i: (0, 0)),
            pl.BlockSpec((NIO * D, D), lambda i: (0, 0)),
            pl.BlockSpec((1, D), lambda i: (0, 0)),
            pl.BlockSpec((D, D), lambda i: (0, 0)),
            pl.BlockSpec((1, D), lambda i: (0, 0)),
            pl.BlockSpec((D, B), lambda i: (0, 0)),
            pl.BlockSpec((1, 1), lambda i: (0, 0)),
            pl.BlockSpec((D, D), lambda i: (0, 0)),
            pl.BlockSpec((1, D), lambda i: (0, 0)),
            pl.BlockSpec((D, 2), lambda i: (0, 0)),
            pl.BlockSpec((1, 2), lambda i: (0, 0)),
        ],
        out_specs=pl.BlockSpec((B, 2), lambda i: (0, 0)),
        out_shape=jax.ShapeDtypeStruct((B, 2), jnp.float32),
        scratch_shapes=[
            pltpu.VMEM((B, D), jnp.float32),
            pltpu.VMEM((1, B), jnp.float32),
            pltpu.VMEM((1, B), jnp.float32),
            pltpu.VMEM((B, D), jnp.float32),
        ],
    )(seg3, flat, extra, W1a, W1b, b1.reshape(1, D), W2, b2.reshape(1, D),
      jnp.broadcast_to(W3, (D, B)), b3.reshape(1, 1), Wf1, bf1.reshape(1, D),
      Wf2, bf2.reshape(1, 2))
    return out
